# baseline (speedup 1.0000x reference)
import functools

import jax
import jax.numpy as jnp
from jax.experimental import pallas as pl
from jax.experimental.pallas import tpu as pltpu

N = 10000
E = 160000
D = 2064
K = 30
TOT = 1025


def _head_body(sp_ref, Wc1_ref, bc1_ref, Wc2r_ref, bc2_ref, Wm1_ref, bm1_ref,
               Wm2_ref, bm2_ref, out_ref):
    sp = sp_ref[...]
    c1 = jnp.dot(sp, Wc1_ref[...].T, preferred_element_type=jnp.float32)
    c1 = jnp.maximum(c1 + bc1_ref[...][None, :], 0.0)
    c1p = jnp.max(c1.reshape(K // 2, 2, 1024), axis=1)
    u = jnp.concatenate([c1p[t:t + 11, :] for t in range(5)], axis=1)
    c2 = jnp.dot(u, Wc2r_ref[...].T, preferred_element_type=jnp.float32)
    c2 = jnp.maximum(c2 + bc2_ref[...][None, :], 0.0)
    pooled = jnp.max(c2, axis=0)
    feat = jnp.maximum(pooled, 0.0)[None, :]
    h1 = jnp.dot(feat, Wm1_ref[...].T, preferred_element_type=jnp.float32)
    h1 = jnp.maximum(h1 + bm1_ref[...][None, :], 0.0)
    logits = jnp.dot(h1, Wm2_ref[...].T, preferred_element_type=jnp.float32)
    logits = logits + bm2_ref[...][None, :]
    out_ref[...] = logits - jax.scipy.special.logsumexp(logits, axis=1, keepdims=True)


def _head(sp, Wc1, bc1, Wc2, bc2, Wm1, bm1, Wm2, bm2):
    Wc2r = Wc2.transpose(0, 2, 1).reshape(2048, 5 * 1024)
    return pl.pallas_call(
        _head_body,
        out_shape=jax.ShapeDtypeStruct((1, 2), jnp.float32),
    )(sp, Wc1, bc1, Wc2r, bc2, Wm1, bm1, Wm2, bm2)


def kernel(node_feat, edge_index, Wg0, bg0, Wg1, bg1, Wg2, bg2,
           Wc1, bc1, Wc2, bc2, Wm1, bm1, Wm2, bm2):
    src = edge_index[0]
    dst = edge_index[1]
    degs = jnp.zeros((N,), jnp.float32).at[dst].add(1.0)
    node_degs = (degs + 1.0)[:, None]
    h = node_feat
    cats = []
    for W, b in ((Wg0, bg0), (Wg1, bg1), (Wg2, bg2)):
        agg = jax.ops.segment_sum(h[src], dst, num_segments=N) + h
        lin = agg @ W.T + b
        h = jnp.tanh(lin / node_degs)
        cats.append(h)
    cur = jnp.concatenate(cats, axis=1)
    to_sort = cur[:, -1]
    _, top_idx = jax.lax.top_k(to_sort, K)
    sp = cur[top_idx]
    return _head(sp, Wc1, bc1, Wc2, bc2, Wm1, bm1, Wm2, bm2)


# speedup vs baseline: 5.2363x; 5.2363x over previous
import functools

import jax
import jax.numpy as jnp
from jax import lax
from jax.experimental import pallas as pl
from jax.experimental.pallas import tpu as pltpu
from jax.experimental.pallas import tpu_sc as plsc

N = 10000
NP = 10240
E = 160000
D = 2064
K = 30
NS = 16
SHARE = NP // NS
EB = 200
EBS = 200
RB = 1024

_f32 = jnp.float32
_i32 = jnp.int32


def _mesh():
    return plsc.VectorSubcoreMesh(core_axis_name="c", subcore_axis_name="s")


def _sc_degs(dst):
    @functools.partial(
        pl.kernel,
        out_type=jax.ShapeDtypeStruct((2, NP), _f32),
        mesh=_mesh(),
        scratch_types=[
            pltpu.VMEM_SHARED((NP,), _f32),
            pltpu.VMEM((EBS,), _i32),
            pltpu.VMEM((EBS,), _f32),
            pltpu.VMEM((SHARE,), _f32),
        ],
    )
    def k(dst_hbm, out_hbm, acc_sh, didx_v, ones_v, zb_v):
        cid = lax.axis_index("c")
        sid = lax.axis_index("s")

        @pl.loop(0, SHARE, step=16)
        def _(i):
            zb_v[pl.ds(i, 16)] = jnp.zeros((16,), _f32)

        @pl.loop(0, EBS, step=16)
        def _(i):
            ones_v[pl.ds(i, 16)] = jnp.full((16,), 1.0, _f32)

        pltpu.sync_copy(zb_v, acc_sh.at[pl.ds(sid * SHARE, SHARE)])
        plsc.subcore_barrier()
        base = cid * (E // 2) + sid * (E // 2 // NS)

        @pl.loop(0, E // 2 // NS, step=EBS)
        def _(eo):
            pltpu.sync_copy(dst_hbm.at[pl.ds(base + eo, EBS)], didx_v)
            pltpu.sync_copy(ones_v, acc_sh.at[didx_v], add=True)

        plsc.subcore_barrier()
        pltpu.sync_copy(acc_sh.at[pl.ds(sid * SHARE, SHARE)],
                        out_hbm.at[cid, pl.ds(sid * SHARE, SHARE)])

    return k(dst)


def _sc_segsum1(src, dst, vals):
    @functools.partial(
        pl.kernel,
        out_type=jax.ShapeDtypeStruct((2, NP), _f32),
        mesh=_mesh(),
        scratch_types=[
            pltpu.VMEM_SHARED((NP,), _f32),
            pltpu.VMEM((EBS,), _i32),
            pltpu.VMEM((EBS,), _i32),
            pltpu.VMEM((EBS,), _f32),
            pltpu.VMEM((SHARE,), _f32),
            pltpu.SemaphoreType.DMA,
        ],
    )
    def k(src_hbm, dst_hbm, v_hbm, out_hbm, acc_sh, sidx_v, didx_v, vals_v, zb_v, sem):
        cid = lax.axis_index("c")
        sid = lax.axis_index("s")

        @pl.loop(0, SHARE, step=16)
        def _(i):
            zb_v[pl.ds(i, 16)] = jnp.zeros((16,), _f32)

        pltpu.sync_copy(zb_v, acc_sh.at[pl.ds(sid * SHARE, SHARE)])
        plsc.subcore_barrier()
        base = cid * (E // 2) + sid * (E // 2 // NS)

        @pl.loop(0, E // 2 // NS, step=EBS)
        def _(eo):
            pltpu.sync_copy(src_hbm.at[pl.ds(base + eo, EBS)], sidx_v)
            pltpu.sync_copy(dst_hbm.at[pl.ds(base + eo, EBS)], didx_v)
            pltpu.async_copy(v_hbm.at[sidx_v], vals_v, sem).wait()
            pltpu.sync_copy(vals_v, acc_sh.at[didx_v], add=True)

        plsc.subcore_barrier()
        pltpu.sync_copy(acc_sh.at[pl.ds(sid * SHARE, SHARE)],
                        out_hbm.at[cid, pl.ds(sid * SHARE, SHARE)])

    return k(src, dst, vals)


def _sc_spmm(src, dst, zc0, zc1, zc2, zc3):
    out_t = [jax.ShapeDtypeStruct((NP, 128), _f32) for _ in range(4)]

    @functools.partial(
        pl.kernel,
        out_type=out_t,
        mesh=_mesh(),
        scratch_types=[
            pltpu.VMEM_SHARED((NP, 128), _f32),
            pltpu.VMEM((EB,), _i32),
            pltpu.VMEM((EB,), _i32),
            pltpu.VMEM((EB, 128), _f32),
            pltpu.SemaphoreType.DMA,
        ],
    )
    def k(src_hbm, dst_hbm, z0, z1, z2, z3, o0, o1, o2, o3,
          acc_sh, sidx_v, didx_v, rows_v, sem):
        cid = lax.axis_index("c")
        sid = lax.axis_index("s")
        zs = (z0, z1, z2, z3)
        os_ = (o0, o1, o2, o3)
        r0 = sid * SHARE
        for chunk in range(4):
            @pl.when(cid == chunk // 2)
            def _(chunk=chunk):
                z_hbm = zs[chunk]
                o_hbm = os_[chunk]
                pltpu.sync_copy(z_hbm.at[pl.ds(r0, SHARE)], acc_sh.at[pl.ds(r0, SHARE)])
                plsc.subcore_barrier()
                ebase = sid * (E // NS)

                @pl.loop(0, E // NS, step=EB)
                def _(eo):
                    pltpu.sync_copy(src_hbm.at[pl.ds(ebase + eo, EB)], sidx_v)
                    pltpu.sync_copy(dst_hbm.at[pl.ds(ebase + eo, EB)], didx_v)
                    pltpu.async_copy(z_hbm.at[sidx_v], rows_v, sem).wait()
                    pltpu.sync_copy(rows_v, acc_sh.at[didx_v], add=True)

                plsc.subcore_barrier()
                pltpu.sync_copy(acc_sh.at[pl.ds(r0, SHARE)], o_hbm.at[pl.ds(r0, SHARE)])
                plsc.subcore_barrier()

    return k(src, dst, zc0, zc1, zc2, zc3)


def _m0_body(x_ref, w_ref, o0, o1, o2, o3):
    z = jnp.dot(x_ref[...], w_ref[...], preferred_element_type=_f32,
                precision=lax.Precision.HIGHEST)
    for c, o in enumerate((o0, o1, o2, o3)):
        o[...] = z[:, c * 128:(c + 1) * 128]


def _m0(xp, w):
    return pl.pallas_call(
        _m0_body,
        grid=(NP // RB,),
        in_specs=[
            pl.BlockSpec((RB, D), lambda i: (i, 0)),
            pl.BlockSpec((D, 512), lambda i: (0, 0)),
        ],
        out_specs=[pl.BlockSpec((RB, 128), lambda i: (i, 0)) for _ in range(4)],
        out_shape=[jax.ShapeDtypeStruct((NP, 128), _f32) for _ in range(4)],
    )(xp, w)


def _m1_body(a0, a1, a2, a3, dp_ref, b_ref, w_ref, h_ref, o0, o1, o2, o3):
    lin = jnp.concatenate([a[...] for a in (a0, a1, a2, a3)], axis=1) + b_ref[...]
    deg = dp_ref[0] + dp_ref[1] + 1.0
    h = jnp.tanh(lin / deg)
    h_ref[...] = h
    z = jnp.dot(h, w_ref[...], preferred_element_type=_f32)
    for c, o in enumerate((o0, o1, o2, o3)):
        o[...] = z[:, c * 128:(c + 1) * 128]


def _m1(aggc, degp, b, w):
    return pl.pallas_call(
        _m1_body,
        grid=(NP // RB,),
        in_specs=[pl.BlockSpec((RB, 128), lambda i: (i, 0)) for _ in range(4)]
        + [
            pl.BlockSpec((2, RB, 1), lambda i: (0, i, 0)),
            pl.BlockSpec((1, 512), lambda i: (0, 0)),
            pl.BlockSpec((512, 512), lambda i: (0, 0)),
        ],
        out_specs=[pl.BlockSpec((RB, 512), lambda i: (i, 0))]
        + [pl.BlockSpec((RB, 128), lambda i: (i, 0)) for _ in range(4)],
        out_shape=[jax.ShapeDtypeStruct((NP, 512), _f32)]
        + [jax.ShapeDtypeStruct((NP, 128), _f32) for _ in range(4)],
    )(*aggc, degp, b, w)


def _m2_body(a0, a1, a2, a3, dp_ref, b_ref, w2_ref, h_ref, z2_ref):
    lin = jnp.concatenate([a[...] for a in (a0, a1, a2, a3)], axis=1) + b_ref[...]
    deg = dp_ref[0] + dp_ref[1] + 1.0
    h = jnp.tanh(lin / deg)
    h_ref[...] = h
    z2_ref[...] = jnp.dot(h, w2_ref[...], preferred_element_type=_f32)


def _m2(aggc, degp, b, w2):
    return pl.pallas_call(
        _m2_body,
        grid=(NP // RB,),
        in_specs=[pl.BlockSpec((RB, 128), lambda i: (i, 0)) for _ in range(4)]
        + [
            pl.BlockSpec((2, RB, 1), lambda i: (0, i, 0)),
            pl.BlockSpec((1, 512), lambda i: (0, 0)),
            pl.BlockSpec((512, 1), lambda i: (0, 0)),
        ],
        out_specs=[
            pl.BlockSpec((RB, 512), lambda i: (i, 0)),
            pl.BlockSpec((RB, 1), lambda i: (i, 0)),
        ],
        out_shape=[
            jax.ShapeDtypeStruct((NP, 512), _f32),
            jax.ShapeDtypeStruct((NP, 1), _f32),
        ],
    )(*aggc, degp, b, w2)


def _t4_body(s3p_ref, z2p_ref, dgp_ref, bg2_ref, h1_ref, h2_ref,
             sp1_ref, sp2_ref, sp3_ref):
    lin = s3p_ref[0] + s3p_ref[1] + z2p_ref[...] + bg2_ref[0, 0]
    deg = dgp_ref[0] + dgp_ref[1] + 1.0
    u = jnp.tanh(lin / deg)
    li = (128 * lax.broadcasted_iota(_i32, (80, 128), 0)
          + lax.broadcasted_iota(_i32, (80, 128), 1))
    u = jnp.where(li < N, u, -3e38)
    kiota = lax.broadcasted_iota(_i32, (1, 32), 1)

    def step(kk, carry):
        u_c, idxs, vals = carry
        m = jnp.max(u_c)
        i = jnp.min(jnp.where(u_c == m, li, jnp.int32(2**31 - 1)))
        idxs = jnp.where(kiota == kk, i, idxs)
        vals = jnp.where(kiota == kk, m, vals)
        u_c = jnp.where(li == i, -3e38, u_c)
        return u_c, idxs, vals

    _, idxs, vals = lax.fori_loop(
        0, K, step,
        (u, jnp.zeros((1, 32), _i32), jnp.zeros((1, 32), _f32)))
    sp3_ref[...] = vals
    sp1_ref[...] = jnp.zeros((32, 512), _f32)
    sp2_ref[...] = jnp.zeros((32, 512), _f32)
    for kk in range(K):
        ik = idxs[0, kk]
        sp1_ref[pl.ds(kk, 1), :] = h1_ref[pl.ds(ik, 1), :]
        sp2_ref[pl.ds(kk, 1), :] = h2_ref[pl.ds(ik, 1), :]


def _t4(s3p, z2p, dgp, bg2, h1, h2):
    return pl.pallas_call(
        _t4_body,
        out_shape=[
            jax.ShapeDtypeStruct((32, 512), _f32),
            jax.ShapeDtypeStruct((32, 512), _f32),
            jax.ShapeDtypeStruct((1, 32), _f32),
        ],
    )(s3p, z2p, dgp, bg2, h1, h2)


def _t5_body(sp1_ref, sp2_ref, sp3_ref, Wc1_ref, bc1_ref, Wc2r_ref, bc2_ref,
             Wm1_ref, bm1_ref, Wm2_ref, bm2_ref, out_ref):
    w = Wc1_ref[...]
    c1 = (jnp.dot(sp1_ref[...][:K], w[:, :512].T, preferred_element_type=_f32)
          + jnp.dot(sp2_ref[...][:K], w[:, 512:1024].T, preferred_element_type=_f32)
          + jnp.dot(sp3_ref[...][:K], w[:, 1024:1025].T, preferred_element_type=_f32))
    c1 = jnp.maximum(c1 + bc1_ref[...][None, :], 0.0)
    c1p = jnp.max(c1.reshape(K // 2, 2, 1024), axis=1)
    u = jnp.concatenate([c1p[t:t + 11, :] for t in range(5)], axis=1)
    c2 = jnp.dot(u, Wc2r_ref[...].T, preferred_element_type=_f32)
    c2 = jnp.maximum(c2 + bc2_ref[...][None, :], 0.0)
    feat = jnp.maximum(jnp.max(c2, axis=0), 0.0)[None, :]
    h1 = jnp.dot(feat, Wm1_ref[...].T, preferred_element_type=_f32)
    h1 = jnp.maximum(h1 + bm1_ref[...][None, :], 0.0)
    logits = jnp.dot(h1, Wm2_ref[...].T, preferred_element_type=_f32)
    logits = logits + bm2_ref[...][None, :]
    out_ref[...] = logits - jax.scipy.special.logsumexp(logits, axis=1, keepdims=True)


def _t5(sp1, sp2, sp3, Wc1, bc1, Wc2, bc2, Wm1, bm1, Wm2, bm2):
    Wc2r = Wc2.transpose(0, 2, 1).reshape(2048, 5 * 1024)
    return pl.pallas_call(
        _t5_body,
        out_shape=jax.ShapeDtypeStruct((1, 2), _f32),
    )(sp1, sp2, sp3, Wc1, bc1, Wc2r, bc2, Wm1, bm1, Wm2, bm2)


def kernel(node_feat, edge_index, Wg0, bg0, Wg1, bg1, Wg2, bg2,
           Wc1, bc1, Wc2, bc2, Wm1, bm1, Wm2, bm2):
    src = edge_index[0]
    dst = edge_index[1]
    xp = jnp.pad(node_feat, ((0, NP - N), (0, 0)))

    degp = _sc_degs(dst)
    z0c = _m0(xp, Wg0.T)
    agg0 = _sc_spmm(src, dst, *z0c)
    h1, *z1c = _m1(agg0, degp.reshape(2, NP, 1), bg0.reshape(1, 512), Wg1.T)
    agg1 = _sc_spmm(src, dst, *z1c)
    h2, z2 = _m2(agg1, degp.reshape(2, NP, 1), bg1.reshape(1, 512), Wg2.T)
    s3p = _sc_segsum1(src, dst, z2.reshape(NP))
    sp1, sp2, sp3 = _t4(
        s3p.reshape(2, 80, 128), z2.reshape(80, 128), degp.reshape(2, 80, 128),
        bg2.reshape(1, 1), h1, h2)
    return _t5(sp1, sp2, sp3.T, Wc1, bc1, Wc2, bc2, Wm1, bm1, Wm2, bm2)
